# Initial kernel scaffold; baseline (speedup 1.0000x reference)
import jax
import jax.numpy as jnp
from jax.experimental import pallas as pl


def kernel(x_1, x_2, n_1, n_2, edge_index, edge_attr, batch_index, params):
    raise NotImplementedError("write your pallas kernel here")


# speedup vs baseline: 3.4429x; 3.4429x over previous
import functools

import jax
import jax.numpy as jnp
from jax import lax
from jax.experimental import pallas as pl
from jax.experimental.pallas import tpu as pltpu
from jax.experimental.pallas import tpu_sc as plsc

N1 = 8192
N2 = 8192
N = N1 + N2
E = 262144
F = 256
H = 256
GOUT = 128
NB = 16
SEG = 512
NT = 32
WE = 2048
NW = E // WE
CAP = 160

_EPS = 1e-5


def _front_body(x1_ref, x2_ref, p1_ref, p2_ref, w0_ref, t0_ref):
    def seq(x, ps):
        for i in range(2):
            w, b, g, be = ps[4 * i], ps[4 * i + 1], ps[4 * i + 2], ps[4 * i + 3]
            x = jnp.dot(x, w[...], preferred_element_type=jnp.float32) + b[...]
            m = jnp.mean(x, axis=0, keepdims=True)
            v = jnp.mean((x - m) ** 2, axis=0, keepdims=True)
            x = (x - m) * lax.rsqrt(v + _EPS) * g[...] + be[...]
            x = jnp.maximum(x, 0.0)
        return x

    w0 = w0_ref[...]
    h1 = jnp.dot(seq(x1_ref[...], p1_ref), w0, preferred_element_type=jnp.float32)
    h2 = jnp.dot(seq(x2_ref[...], p2_ref), w0, preferred_element_type=jnp.float32)
    for k in range(NB):
        t0_ref[pl.ds(2 * SEG * k, SEG), :] = h1[SEG * k:SEG * (k + 1), :]
        t0_ref[pl.ds(2 * SEG * k + SEG, SEG), :] = h2[SEG * k:SEG * (k + 1), :]


def _front(x1, x2, t1p, t2p, w0):
    vec = lambda a: a.reshape(1, -1)
    p1 = [t1p[0]["W"], vec(t1p[0]["b"]), vec(t1p[0]["g"]), vec(t1p[0]["beta"]),
          t1p[1]["W"], vec(t1p[1]["b"]), vec(t1p[1]["g"]), vec(t1p[1]["beta"])]
    p2 = [t2p[0]["W"], vec(t2p[0]["b"]), vec(t2p[0]["g"]), vec(t2p[0]["beta"]),
          t2p[1]["W"], vec(t2p[1]["b"]), vec(t2p[1]["g"]), vec(t2p[1]["beta"])]
    return pl.pallas_call(
        _front_body,
        out_shape=jax.ShapeDtypeStruct((N, H), jnp.float32),
        compiler_params=pltpu.CompilerParams(
            vmem_limit_bytes=128 * 1024 * 1024),
    )(x1, x2, p1, p2, w0)


def _b1_body(dst_hbm, w_hbm, src_hbm, deg_hbm, wraw_hbm,
             dwin, wwin, hist, wrow, teid, sbuf, wbuf, degbuf, sem):
    wid = lax.axis_index("s") * 2 + lax.axis_index("c")
    base = wid * SEG
    lane = lax.broadcasted_iota(jnp.int32, (16,), 0)
    zero = jnp.zeros((16,), jnp.float32)

    def loop(n, body):
        lax.fori_loop(0, n, lambda i, c: (body(i), 0)[1], 0)

    loop(SEG, lambda r: [hist.__setitem__(pl.ds(r * 16, 16), zero)])
    loop(N // 16, lambda r: [wrow.__setitem__(pl.ds(r * 16, 16), zero)])
    izero = jnp.zeros((16,), jnp.int32)
    for r in range(CAP // 16):
        teid[pl.ds(r * 16, 16)] = izero
    for r in range(144 // 16):
        sbuf[pl.ds(r * 16, 16)] = izero
        wbuf[pl.ds(r * 16, 16)] = zero

    def drain(off, width, count):
        @pl.when(count > 0)
        def _():
            pltpu.async_copy(src_hbm.at[teid.at[pl.ds(off, width)]],
                             sbuf.at[pl.ds(0, width)], sem).wait()
            pltpu.async_copy(w_hbm.at[teid.at[pl.ds(off, width)]],
                             wbuf.at[pl.ds(0, width)], sem).wait()

            lane0 = lax.broadcasted_iota(jnp.int32, (16,), 0) == 0

            def upd(j, c):
                sv = sbuf[pl.ds(j, 16)]
                wv = wbuf[pl.ds(j, 16)]
                plsc.addupdate_scatter(wrow, [sv], wv, mask=lane0)
                return c
            lax.fori_loop(0, count, upd, 0)

    def window(win, cnt2):
        pltpu.sync_copy(dst_hbm.at[pl.ds(win * WE, WE)], dwin)
        pltpu.sync_copy(w_hbm.at[pl.ds(win * WE, WE)], wwin)

        def vec(v, cnt2):
            dv = dwin[pl.ds(v * 16, 16)]
            rel = dv - base
            m = (rel >= 0) & (rel < SEG)
            relc = jnp.where(m, rel, 0)
            plsc.addupdate_scatter(hist, [lane * SEG + relc],
                                   wwin[pl.ds(v * 16, 16)], mask=m)
            m2 = dv == base
            pc2 = lax.reduce_sum(jnp.where(m2, 1, 0), axes=(0,))

            def app(cnt2):
                eid = win * WE + v * 16 + lane
                plsc.store_compressed(teid.at[pl.ds(cnt2, 16)], eid, mask=m2)
                cnt2 = cnt2 + pc2

                def flush(cnt2):
                    drain(0, 128, 128)
                    teid[pl.ds(0, 16)] = teid[pl.ds(128, 16)]
                    teid[pl.ds(16, 16)] = teid[pl.ds(144, 16)]
                    return cnt2 - 128
                return lax.cond(cnt2 >= 128, flush, lambda c: c, cnt2)
            return lax.cond(pc2 > 0, app, lambda c: c, cnt2)
        return lax.fori_loop(0, WE // 16, vec, cnt2)

    cnt2 = lax.fori_loop(0, NW, window, 0)
    drain(0, 128, jnp.minimum(cnt2, 128))
    drain(128, 32, jnp.maximum(cnt2 - 128, 0))

    def red(g, c):
        acc = hist[pl.ds(g * 16, 16)]
        for l in range(1, 16):
            acc = acc + hist[pl.ds(l * SEG + g * 16, 16)]
        degbuf[pl.ds(g * 16, 16)] = acc
        return c
    lax.fori_loop(0, SEG // 16, red, 0)
    pltpu.sync_copy(degbuf, deg_hbm.at[pl.ds(base, SEG)])
    pltpu.sync_copy(wrow, wraw_hbm.at[wid])


def _b1(dst, w, src):
    return pl.kernel(
        _b1_body,
        out_type=[jax.ShapeDtypeStruct((N,), jnp.float32),
                  jax.ShapeDtypeStruct((NT, N), jnp.float32)],
        mesh=plsc.VectorSubcoreMesh(core_axis_name="c", subcore_axis_name="s"),
        compiler_params=pltpu.CompilerParams(needs_layout_passes=False),
        scratch_types=[
            pltpu.VMEM((WE,), jnp.int32),
            pltpu.VMEM((WE,), jnp.float32),
            pltpu.VMEM((16 * SEG,), jnp.float32),
            pltpu.VMEM((N,), jnp.float32),
            pltpu.VMEM((CAP,), jnp.int32),
            pltpu.VMEM((144,), jnp.int32),
            pltpu.VMEM((144,), jnp.float32),
            pltpu.VMEM((SEG,), jnp.float32),
            pltpu.SemaphoreType.DMA,
        ],
    )(dst, w, src)


def _dis_body(deg_ref, dis_ref, dt_ref):
    deg = deg_ref[...]
    dis = jnp.where(deg > 0, lax.rsqrt(jnp.where(deg > 0, deg, 1.0)), 0.0)
    dis_ref[...] = dis
    dt_ref[...] = jnp.reshape(dis, (NT, (N // NT) // 128, 128))[:, 0, 0:1]


def _dis(deg2d):
    return pl.pallas_call(
        _dis_body,
        out_shape=[jax.ShapeDtypeStruct((128, 128), jnp.float32),
                   jax.ShapeDtypeStruct((NT, 1), jnp.float32)],
    )(deg2d)


def _b2_body(dst_hbm, src_hbm, w_hbm, dis_hbm, t0_hbm, out_hbm,
             dwin, disl, acc, meid, mrel, sbuf, wbuf, nbuf, rows, sem):
    wid = lax.axis_index("s") * 2 + lax.axis_index("c")
    lane = lax.broadcasted_iota(jnp.int32, (16,), 0)
    zero = jnp.zeros((16,), jnp.float32)
    C = 256

    pltpu.sync_copy(dis_hbm, disl)
    izero = jnp.zeros((16,), jnp.int32)
    for r in range(CAP // 16):
        meid[pl.ds(r * 16, 16)] = izero
        mrel[pl.ds(r * 16, 16)] = izero
        nbuf[pl.ds(r * 16, 16)] = zero
    for r in range(144 // 16):
        sbuf[pl.ds(r * 16, 16)] = izero
        wbuf[pl.ds(r * 16, 16)] = zero

    for p in range(2):
        base = (p * NT + wid) * C

        def zrow(r, c):
            for q in range(16):
                acc[r, pl.ds(q * 16, 16)] = zero
            return c
        lax.fori_loop(0, C, zrow, 0)

        def flush(off, width, count):
            @pl.when(count > 0)
            def _():
                pltpu.async_copy(src_hbm.at[meid.at[pl.ds(off, width)]],
                                 sbuf.at[pl.ds(0, width)], sem).wait()
                pltpu.async_copy(w_hbm.at[meid.at[pl.ds(off, width)]],
                                 wbuf.at[pl.ds(0, width)], sem).wait()
                for g in range(width // 16):
                    sv = sbuf[pl.ds(g * 16, 16)]
                    rv = mrel[pl.ds(off + g * 16, 16)]
                    nv = (plsc.load_gather(disl, [sv])
                          * wbuf[pl.ds(g * 16, 16)]
                          * plsc.load_gather(disl, [rv + base]))
                    nbuf[pl.ds(g * 16, 16)] = nv
                pltpu.async_copy(t0_hbm.at[sbuf.at[pl.ds(0, width)]],
                                 rows.at[pl.ds(0, width)], sem).wait()

                def upd(j, c):
                    d = mrel[pl.ds(off + j, 16)][0]
                    n = nbuf[pl.ds(j, 16)][0]
                    for q in range(16):
                        plsc.addupdate(acc.at[d, pl.ds(q * 16, 16)],
                                       n * rows[j, pl.ds(q * 16, 16)])
                    return c
                lax.fori_loop(0, count, upd, 0)

        def window(win, cnt):
            pltpu.sync_copy(dst_hbm.at[pl.ds(win * WE, WE)], dwin)

            def vec(v, cnt):
                dv = dwin[pl.ds(v * 16, 16)]
                rel = dv - base
                m = (rel >= 0) & (rel < C)
                pc = lax.reduce_sum(jnp.where(m, 1, 0), axes=(0,))

                def app(cnt):
                    eid = win * WE + v * 16 + lane
                    plsc.store_compressed(meid.at[pl.ds(cnt, 16)], eid, mask=m)
                    plsc.store_compressed(mrel.at[pl.ds(cnt, 16)],
                                          jnp.where(m, rel, 0), mask=m)
                    cnt = cnt + pc

                    def fl(cnt):
                        flush(0, 128, 128)
                        meid[pl.ds(0, 16)] = meid[pl.ds(128, 16)]
                        meid[pl.ds(16, 16)] = meid[pl.ds(144, 16)]
                        mrel[pl.ds(0, 16)] = mrel[pl.ds(128, 16)]
                        mrel[pl.ds(16, 16)] = mrel[pl.ds(144, 16)]
                        return cnt - 128
                    return lax.cond(cnt >= 128, fl, lambda c: c, cnt)
                return lax.cond(pc > 0, app, lambda c: c, cnt)
            return lax.fori_loop(0, WE // 16, vec, cnt)

        cnt = lax.fori_loop(0, NW, window, 0)
        flush(0, 128, jnp.minimum(cnt, 128))
        flush(128, 32, jnp.maximum(cnt - 128, 0))
        pltpu.sync_copy(acc, out_hbm.at[pl.ds(base, C)])


def _b2(dst, src, w, dis, t0):
    return pl.kernel(
        _b2_body,
        out_type=jax.ShapeDtypeStruct((N, H), jnp.float32),
        mesh=plsc.VectorSubcoreMesh(core_axis_name="c", subcore_axis_name="s"),
        compiler_params=pltpu.CompilerParams(needs_layout_passes=False),
        scratch_types=[
            pltpu.VMEM((WE,), jnp.int32),
            pltpu.VMEM((N,), jnp.float32),
            pltpu.VMEM((256, H), jnp.float32),
            pltpu.VMEM((CAP,), jnp.int32),
            pltpu.VMEM((CAP,), jnp.int32),
            pltpu.VMEM((144,), jnp.int32),
            pltpu.VMEM((144,), jnp.float32),
            pltpu.VMEM((160,), jnp.float32),
            pltpu.VMEM((128, H), jnp.float32),
            pltpu.SemaphoreType.DMA,
        ],
    )(dst, src, w, dis, t0)


def _back_body(out0_ref, b0_ref, g0_ref, be0_ref, wraw_ref, dis_ref,
               dt_ref, w1_ref, b1_ref, o1_ref, o2_ref):
    h = out0_ref[...] + b0_ref[...]
    m = jnp.mean(h, axis=0, keepdims=True)
    v = jnp.mean((h - m) ** 2, axis=0, keepdims=True)
    t1 = jnp.maximum((h - m) * lax.rsqrt(v + _EPS) * g0_ref[...] + be0_ref[...],
                     0.0)
    wm = wraw_ref[...] * dis_ref[...]
    o32 = jnp.dot(wm, t1, preferred_element_type=jnp.float32)
    o = dt_ref[...] * jnp.dot(o32, w1_ref[...],
                              preferred_element_type=jnp.float32) + b1_ref[...]
    o = jnp.reshape(o, (NB, 2, GOUT))
    o1_ref[...] = o[:, 0, :]
    o2_ref[...] = o[:, 1, :]


def _back(out0, b0, g0, be0, wraw, dis_row, dt, w1, b1):
    return pl.pallas_call(
        _back_body,
        out_shape=[jax.ShapeDtypeStruct((NB, GOUT), jnp.float32),
                   jax.ShapeDtypeStruct((NB, GOUT), jnp.float32)],
        compiler_params=pltpu.CompilerParams(
            vmem_limit_bytes=128 * 1024 * 1024),
    )(out0, b0, g0, be0, wraw, dis_row, dt, w1, b1)


def kernel(x_1, x_2, n_1, n_2, edge_index, edge_attr, batch_index, params):
    src = edge_index[0]
    dst = edge_index[1]
    w = edge_attr[:, 0]
    gcn = params["gcn"]

    t0 = _front(x_1, x_2, params["t1"], params["t2"], gcn["W0"])
    deg, wraw = _b1(dst, w, src)
    dis2d, dt = _dis(deg.reshape(128, 128))
    dis_flat = dis2d.reshape(N)
    out0 = _b2(dst, src, w, dis_flat, t0)
    vec = lambda a: a.reshape(1, -1)
    o1, o2 = _back(out0, vec(gcn["b0"]), vec(gcn["g0"]), vec(gcn["beta0"]),
                   wraw, dis_flat.reshape(1, N), dt, gcn["W1"], vec(gcn["b1"]))
    return (o1, o2)


# speedup vs baseline: 3.8727x; 1.1248x over previous
import functools

import jax
import jax.numpy as jnp
from jax import lax
from jax.experimental import pallas as pl
from jax.experimental.pallas import tpu as pltpu
from jax.experimental.pallas import tpu_sc as plsc

N1 = 8192
N2 = 8192
N = N1 + N2
E = 262144
F = 256
H = 256
GOUT = 128
NB = 16
SEG = 512
NT = 32
WE = 2048
NW = E // WE
CAP = 160

_EPS = 1e-5


def _front_body(x1_ref, x2_ref, p1_ref, p2_ref, w0_ref, t0_ref):
    def seq(x, ps):
        for i in range(2):
            w, b, g, be = ps[4 * i], ps[4 * i + 1], ps[4 * i + 2], ps[4 * i + 3]
            x = jnp.dot(x, w[...], preferred_element_type=jnp.float32) + b[...]
            m = jnp.mean(x, axis=0, keepdims=True)
            v = jnp.mean((x - m) ** 2, axis=0, keepdims=True)
            x = (x - m) * lax.rsqrt(v + _EPS) * g[...] + be[...]
            x = jnp.maximum(x, 0.0)
        return x

    w0 = w0_ref[...]
    h1 = jnp.dot(seq(x1_ref[...], p1_ref), w0, preferred_element_type=jnp.float32)
    h2 = jnp.dot(seq(x2_ref[...], p2_ref), w0, preferred_element_type=jnp.float32)
    for k in range(NB):
        t0_ref[pl.ds(2 * SEG * k, SEG), :] = h1[SEG * k:SEG * (k + 1), :]
        t0_ref[pl.ds(2 * SEG * k + SEG, SEG), :] = h2[SEG * k:SEG * (k + 1), :]


def _front(x1, x2, t1p, t2p, w0):
    vec = lambda a: a.reshape(1, -1)
    p1 = [t1p[0]["W"], vec(t1p[0]["b"]), vec(t1p[0]["g"]), vec(t1p[0]["beta"]),
          t1p[1]["W"], vec(t1p[1]["b"]), vec(t1p[1]["g"]), vec(t1p[1]["beta"])]
    p2 = [t2p[0]["W"], vec(t2p[0]["b"]), vec(t2p[0]["g"]), vec(t2p[0]["beta"]),
          t2p[1]["W"], vec(t2p[1]["b"]), vec(t2p[1]["g"]), vec(t2p[1]["beta"])]
    return pl.pallas_call(
        _front_body,
        out_shape=jax.ShapeDtypeStruct((N, H), jnp.float32),
        compiler_params=pltpu.CompilerParams(
            vmem_limit_bytes=128 * 1024 * 1024),
    )(x1, x2, p1, p2, w0)


EW = E // NT


def _b1_body(dst2_hbm, w_hbm, src_hbm, degp_hbm, wrawp_hbm,
             dwin, wwin, swin, tidx, tval, zbuf, sh_deg, sh_wraw, sem):
    cid = lax.axis_index("c")
    sid = lax.axis_index("s")
    wid = sid * 2 + cid
    lane = lax.broadcasted_iota(jnp.int32, (16,), 0)
    zero = jnp.zeros((16,), jnp.float32)
    izero = jnp.zeros((16,), jnp.int32)

    def loop(n, body):
        lax.fori_loop(0, n, lambda i, c: (body(i), 0)[1], 0)

    loop(4096 // 16, lambda r: [zbuf.__setitem__(pl.ds(r * 16, 16), zero)])
    for r in range(8):
        tidx[0, pl.ds(r * 16, 16)] = izero
        tval[0, pl.ds(r * 16, 16)] = zero

    cpd = pltpu.async_copy(dst2_hbm.at[pl.ds(wid * (EW // 128), EW // 128)],
                           dwin, sem)
    cpw = pltpu.async_copy(w_hbm.at[pl.ds(wid * EW, EW)], wwin, sem)
    cps = pltpu.async_copy(src_hbm.at[pl.ds(wid * EW, EW)], swin, sem)

    pltpu.sync_copy(zbuf.at[pl.ds(0, 1024)], sh_deg.at[pl.ds(sid * 1024, 1024)])
    for r in range(8):
        pltpu.sync_copy(zbuf, sh_wraw.at[pl.ds(sid * 32768 + r * 4096, 4096)])
    plsc.subcore_barrier()
    cpd.wait()
    cpw.wait()
    cps.wait()

    for c in range(EW // 128):
        pltpu.sync_copy(wwin.at[pl.ds(c * 128, 128)],
                        sh_deg.at[dwin.at[c]], add=True)

    def wflush():
        pltpu.sync_copy(tval.at[0], sh_wraw.at[tidx.at[0]], add=True)
        for r in range(8):
            tval[0, pl.ds(r * 16, 16)] = zero

    def vec(v, cnt):
        dv = dwin[v >> 3, pl.ds((v & 7) * 16, 16)]
        m = (dv & 511) == 0

        def app(cnt):
            sv = swin[pl.ds(v * 16, 16)]
            wv = wwin[pl.ds(v * 16, 16)]
            fi = lax.shift_right_logical(dv, 9) * N + sv
            pc = lax.reduce_sum(jnp.where(m, 1, 0), axes=(0,))
            plsc.store_compressed(tidx.at[0, pl.ds(cnt, 16)],
                                  jnp.where(m, fi, 0), mask=m)
            plsc.store_compressed(tval.at[0, pl.ds(cnt, 16)], wv, mask=m)
            cnt = cnt + pc

            def fl(cnt):
                wflush()
                return cnt * 0
            return lax.cond(cnt > 112, fl, lambda c: c, cnt)
        return lax.cond(jnp.any(m), app, lambda c: c, cnt)

    cnt = lax.fori_loop(0, EW // 16, vec, 0)

    @pl.when(cnt > 0)
    def _():
        wflush()
    plsc.subcore_barrier()

    pltpu.sync_copy(sh_deg.at[pl.ds(sid * 1024, 1024)],
                    degp_hbm.at[pl.ds(cid * N + sid * 1024, 1024)])
    for r in range(8):
        off = sid * 32768 + r * 4096
        pltpu.sync_copy(sh_wraw.at[pl.ds(off, 4096)],
                        wrawp_hbm.at[pl.ds(cid * (NT * N) + off, 4096)])


def _b1(dst2, w, src):
    return pl.kernel(
        _b1_body,
        out_type=[jax.ShapeDtypeStruct((2 * N,), jnp.float32),
                  jax.ShapeDtypeStruct((2 * NT * N,), jnp.float32)],
        mesh=plsc.VectorSubcoreMesh(core_axis_name="c", subcore_axis_name="s"),
        compiler_params=pltpu.CompilerParams(needs_layout_passes=False),
        scratch_types=[
            pltpu.VMEM((EW // 128, 128), jnp.int32),
            pltpu.VMEM((EW,), jnp.float32),
            pltpu.VMEM((EW,), jnp.int32),
            pltpu.VMEM((1, 128), jnp.int32),
            pltpu.VMEM((1, 128), jnp.float32),
            pltpu.VMEM((4096,), jnp.float32),
            pltpu.VMEM_SHARED((N,), jnp.float32),
            pltpu.VMEM_SHARED((NT * N,), jnp.float32),
            pltpu.SemaphoreType.DMA,
        ],
    )(dst2, w, src)


def _dis_body(deg_ref, dis_ref, dt_ref):
    degp = deg_ref[...]
    deg = degp[0] + degp[1]
    dis = jnp.where(deg > 0, lax.rsqrt(jnp.where(deg > 0, deg, 1.0)), 0.0)
    dis_ref[...] = dis
    dt_ref[...] = jnp.reshape(dis, (NT, (N // NT) // 128, 128))[:, 0, 0:1]


def _dis(degp):
    return pl.pallas_call(
        _dis_body,
        out_shape=[jax.ShapeDtypeStruct((128, 128), jnp.float32),
                   jax.ShapeDtypeStruct((NT, 1), jnp.float32)],
    )(degp)


WE2 = 4096
NW2 = E // WE2


def _b2_body(dst_hbm, src_hbm, w_hbm, dis_hbm, t0_hbm, out_hbm,
             dwin2, disl, acc, meid, mrel, sbuf, wbuf, nbuf, rows,
             sem, semA, semB):
    wid = lax.axis_index("s") * 2 + lax.axis_index("c")
    lane = lax.broadcasted_iota(jnp.int32, (16,), 0)
    zero = jnp.zeros((16,), jnp.float32)
    C = 256
    cu = jnp.uint32(C)

    pltpu.sync_copy(dis_hbm, disl)
    izero = jnp.zeros((16,), jnp.int32)
    for r in range(CAP // 16):
        meid[pl.ds(r * 16, 16)] = izero
        mrel[pl.ds(r * 16, 16)] = izero
        nbuf[pl.ds(r * 16, 16)] = zero
    for r in range(144 // 16):
        sbuf[pl.ds(r * 16, 16)] = izero
        wbuf[pl.ds(r * 16, 16)] = zero

    def start(w, buf, s):
        pltpu.async_copy(dst_hbm.at[pl.ds(w * WE2, WE2)], dwin2.at[buf], s)

    def wait(buf, s):
        pltpu.make_async_copy(dst_hbm.at[pl.ds(0, WE2)], dwin2.at[buf], s).wait()

    for p in range(2):
        base = (p * NT + wid) * C

        def zrow(r, c):
            for q in range(16):
                acc[r, pl.ds(q * 16, 16)] = zero
            return c
        lax.fori_loop(0, C, zrow, 0)

        def flush(off, width, count):
            @pl.when(count > 0)
            def _():
                cps = pltpu.async_copy(src_hbm.at[meid.at[pl.ds(off, width)]],
                                       sbuf.at[pl.ds(0, width)], sem)
                cpw = pltpu.async_copy(w_hbm.at[meid.at[pl.ds(off, width)]],
                                       wbuf.at[pl.ds(0, width)], sem)
                cps.wait()
                cpw.wait()
                cpr = pltpu.async_copy(t0_hbm.at[sbuf.at[pl.ds(0, width)]],
                                       rows.at[pl.ds(0, width)], sem)
                for g in range(width // 16):
                    sv = sbuf[pl.ds(g * 16, 16)]
                    rv = mrel[pl.ds(off + g * 16, 16)]
                    nv = (plsc.load_gather(disl, [sv])
                          * wbuf[pl.ds(g * 16, 16)]
                          * plsc.load_gather(disl, [rv + base]))
                    nbuf[pl.ds(g * 16, 16)] = nv
                cpr.wait()

                def upd(j, c):
                    d = mrel[pl.ds(off + j, 16)][0]
                    n = nbuf[pl.ds(j, 16)][0]
                    for q in range(16):
                        plsc.addupdate(acc.at[d, pl.ds(q * 16, 16)],
                                       n * rows[j, pl.ds(q * 16, 16)])
                    return c
                lax.fori_loop(0, count, upd, 0)

        def scan(win, buf, cnt):
            def vec(v, cnt):
                dv = dwin2[buf, pl.ds(v * 16, 16)]
                m = plsc.bitcast(dv - base, jnp.uint32) < cu

                def app(cnt):
                    rel = dv - base
                    pc = lax.reduce_sum(jnp.where(m, 1, 0), axes=(0,))
                    eid = win * WE2 + v * 16 + lane
                    plsc.store_compressed(meid.at[pl.ds(cnt, 16)], eid, mask=m)
                    plsc.store_compressed(mrel.at[pl.ds(cnt, 16)],
                                          jnp.where(m, rel, 0), mask=m)
                    cnt = cnt + pc

                    def fl(cnt):
                        flush(0, 128, 128)
                        meid[pl.ds(0, 16)] = meid[pl.ds(128, 16)]
                        meid[pl.ds(16, 16)] = meid[pl.ds(144, 16)]
                        mrel[pl.ds(0, 16)] = mrel[pl.ds(128, 16)]
                        mrel[pl.ds(16, 16)] = mrel[pl.ds(144, 16)]
                        return cnt - 128
                    return lax.cond(cnt >= 128, fl, lambda c: c, cnt)
                return lax.cond(jnp.any(m), app, lambda c: c, cnt)
            return lax.fori_loop(0, WE2 // 16, vec, cnt)

        start(0, 0, semA)

        def pair(i, cnt):
            w0 = 2 * i
            w1 = 2 * i + 1
            wait(0, semA)
            start(w1, 1, semB)
            cnt = scan(w0, 0, cnt)
            wait(1, semB)

            @pl.when(w1 + 1 < NW2)
            def _():
                start(w1 + 1, 0, semA)
            return scan(w1, 1, cnt)

        cnt = lax.fori_loop(0, NW2 // 2, pair, 0)
        flush(0, 128, jnp.minimum(cnt, 128))
        flush(128, 32, jnp.maximum(cnt - 128, 0))
        pltpu.sync_copy(acc, out_hbm.at[pl.ds(base, C)])


def _b2(dst, src, w, dis, t0):
    return pl.kernel(
        _b2_body,
        out_type=jax.ShapeDtypeStruct((N, H), jnp.float32),
        mesh=plsc.VectorSubcoreMesh(core_axis_name="c", subcore_axis_name="s"),
        compiler_params=pltpu.CompilerParams(needs_layout_passes=False),
        scratch_types=[
            pltpu.VMEM((2, WE2), jnp.int32),
            pltpu.VMEM((N,), jnp.float32),
            pltpu.VMEM((256, H), jnp.float32),
            pltpu.VMEM((CAP,), jnp.int32),
            pltpu.VMEM((CAP,), jnp.int32),
            pltpu.VMEM((144,), jnp.int32),
            pltpu.VMEM((144,), jnp.float32),
            pltpu.VMEM((160,), jnp.float32),
            pltpu.VMEM((128, H), jnp.float32),
            pltpu.SemaphoreType.DMA,
            pltpu.SemaphoreType.DMA,
            pltpu.SemaphoreType.DMA,
        ],
    )(dst, src, w, dis, t0)


def _back_body(out0_ref, b0_ref, g0_ref, be0_ref, wraw_ref, dis_ref,
               dt_ref, w1_ref, b1_ref, o1_ref, o2_ref):
    h = out0_ref[...] + b0_ref[...]
    m = jnp.mean(h, axis=0, keepdims=True)
    v = jnp.mean((h - m) ** 2, axis=0, keepdims=True)
    t1 = jnp.maximum((h - m) * lax.rsqrt(v + _EPS) * g0_ref[...] + be0_ref[...],
                     0.0)
    wraw3 = wraw_ref[...]
    wm = (wraw3[0] + wraw3[1]) * dis_ref[...]
    o32 = jnp.dot(wm, t1, preferred_element_type=jnp.float32)
    o = dt_ref[...] * jnp.dot(o32, w1_ref[...],
                              preferred_element_type=jnp.float32) + b1_ref[...]
    o = jnp.reshape(o, (NB, 2, GOUT))
    o1_ref[...] = o[:, 0, :]
    o2_ref[...] = o[:, 1, :]


def _back(out0, b0, g0, be0, wraw, dis_row, dt, w1, b1):
    return pl.pallas_call(
        _back_body,
        out_shape=[jax.ShapeDtypeStruct((NB, GOUT), jnp.float32),
                   jax.ShapeDtypeStruct((NB, GOUT), jnp.float32)],
        compiler_params=pltpu.CompilerParams(
            vmem_limit_bytes=128 * 1024 * 1024),
    )(out0, b0, g0, be0, wraw, dis_row, dt, w1, b1)


def kernel(x_1, x_2, n_1, n_2, edge_index, edge_attr, batch_index, params):
    src = edge_index[0]
    dst = edge_index[1]
    w = edge_attr[:, 0]
    gcn = params["gcn"]

    t0 = _front(x_1, x_2, params["t1"], params["t2"], gcn["W0"])
    degp, wrawp = _b1(dst.reshape(E // 128, 128), w, src)
    dis2d, dt = _dis(degp.reshape(2, 128, 128))
    dis_flat = dis2d.reshape(N)
    out0 = _b2(dst, src, w, dis_flat, t0)
    vec = lambda a: a.reshape(1, -1)
    o1, o2 = _back(out0, vec(gcn["b0"]), vec(gcn["g0"]), vec(gcn["beta0"]),
                   wrawp.reshape(2, NT, N), dis_flat.reshape(1, N), dt,
                   gcn["W1"], vec(gcn["b1"]))
    return (o1, o2)
